# baseline (speedup 1.0000x reference)
import functools
import math

import jax
import jax.numpy as jnp
from jax import lax
from jax.experimental import pallas as pl
from jax.experimental.pallas import tpu as pltpu
from jax.experimental.pallas import tpu_sc as plsc

_D = 64
_CHUNK = 512
_SUB = 128
_LANES = 16


def _rsqrt(x):
    i = plsc.bitcast(x, jnp.int32)
    y = plsc.bitcast(jnp.int32(0x5F3759DF) - (i >> 1), jnp.float32)
    for _ in range(3):
        y = y * (1.5 - 0.5 * x * y * y)
    return y


def _make_sc_kernel(n_rows, seq_len, n_workers):
    rows_per_w = n_rows // n_workers
    chunks_per_w = rows_per_w // _CHUNK
    mesh = plsc.VectorSubcoreMesh(core_axis_name="c", subcore_axis_name="s")

    @functools.partial(
        pl.kernel,
        out_type=jax.ShapeDtypeStruct((n_rows, _D), jnp.float32),
        mesh=mesh,
        compiler_params=pltpu.CompilerParams(
            needs_layout_passes=False, use_tc_tiling_on_sc=False),
        scratch_types=[
            pltpu.VMEM((_CHUNK,), jnp.int32),
            pltpu.VMEM((_CHUNK, _D), jnp.float32),
            pltpu.VMEM((_D, seq_len + _LANES), jnp.float32),
            pltpu.VMEM((_D,), jnp.float32),
            pltpu.VMEM((_D,), jnp.float32),
            pltpu.VMEM((_CHUNK,), jnp.float32),
            pltpu.VMEM((_CHUNK,), jnp.float32),
            pltpu.SemaphoreType.DMA,
        ],
    )
    def sc_kernel(ids_hbm, table_hbm, pet_hbm, w_hbm, b_hbm, out_hbm,
                  idx_v, rows_v, pet_v, w_v, b_v, mean_v, rstd_v, sem):
        wid = lax.axis_index("s") * 2 + lax.axis_index("c")
        row_base = wid * rows_per_w

        pltpu.sync_copy(pet_hbm, pet_v)
        pltpu.sync_copy(w_hbm, w_v)
        pltpu.sync_copy(b_hbm, b_v)

        lane = lax.iota(jnp.int32, _LANES)
        wq = [w_v[pl.ds(q * _LANES, _LANES)] for q in range(_D // _LANES)]
        bq = [b_v[pl.ds(q * _LANES, _LANES)] for q in range(_D // _LANES)]

        def chunk_body(c, _):
            base = row_base + c * _CHUNK

            pltpu.sync_copy(ids_hbm.at[pl.ds(base, _CHUNK)], idx_v)
            copies = [
                pltpu.async_copy(
                    table_hbm.at[idx_v.at[pl.ds(k * _SUB, _SUB)]],
                    rows_v.at[pl.ds(k * _SUB, _SUB)],
                    sem,
                )
                for k in range(_CHUNK // _SUB)
            ]
            for cp in copies:
                cp.wait()

            def group_body(g, _):
                r0 = g * _LANES
                rows16 = r0 + lane
                s0 = (base + r0) % seq_len
                acc = [jnp.zeros((_LANES,), jnp.float32) for _ in range(4)]
                acc2 = [jnp.zeros((_LANES,), jnp.float32) for _ in range(4)]
                for j in range(_D):
                    colj = jnp.full((_LANES,), j, jnp.int32)
                    ve = plsc.load_gather(rows_v, [rows16, colj])
                    vp = pet_v[j, pl.ds(s0, _LANES)]
                    x = ve + vp
                    plsc.store_scatter(rows_v, [rows16, colj], x)
                    acc[j % 4] = acc[j % 4] + x
                    acc2[j % 4] = acc2[j % 4] + x * x
                stot = (acc[0] + acc[1]) + (acc[2] + acc[3])
                s2tot = (acc2[0] + acc2[1]) + (acc2[2] + acc2[3])
                mean = stot * (1.0 / _D)
                var = s2tot * (1.0 / _D) - mean * mean
                rstd = _rsqrt(var + 1e-5)
                plsc.store_scatter(mean_v, [rows16], mean)
                plsc.store_scatter(rstd_v, [rows16], rstd)
                return 0

            lax.fori_loop(0, _CHUNK // _LANES, group_body, 0, unroll=False)

            def group2_body(g, _):
                r0 = g * _LANES
                mean16 = mean_v[pl.ds(r0, _LANES)]
                rstd16 = rstd_v[pl.ds(r0, _LANES)]
                for i in range(_LANES):
                    m = mean16[i]
                    rs = rstd16[i]
                    r = r0 + i
                    for q in range(_D // _LANES):
                        a = rs * wq[q]
                        off = bq[q] - m * a
                        x = rows_v[r, pl.ds(q * _LANES, _LANES)]
                        rows_v[r, pl.ds(q * _LANES, _LANES)] = x * a + off
                return 0

            lax.fori_loop(0, _CHUNK // _LANES, group2_body, 0, unroll=False)

            pltpu.sync_copy(rows_v, out_hbm.at[pl.ds(base, _CHUNK)])
            return 0

        lax.fori_loop(0, chunks_per_w, chunk_body, 0, unroll=False)

    return sc_kernel


@jax.jit
def kernel(input_ids, word_table, pe, ln_weight, ln_bias):
    b, s = input_ids.shape
    n_rows = b * s
    n_workers = 32
    assert n_rows % (n_workers * _CHUNK) == 0

    ids2d = input_ids.reshape(n_rows).astype(jnp.int32)
    pe_s = pe[:s].astype(jnp.float32)
    pe_t = jnp.concatenate([pe_s, pe_s[:_LANES]], axis=0).T

    sc = _make_sc_kernel(n_rows, s, n_workers)
    out = sc(ids2d, word_table.astype(jnp.float32), pe_t,
             ln_weight.astype(jnp.float32), ln_bias.astype(jnp.float32))
    return out.reshape(b, s, _D)
